# baseline (speedup 1.0000x reference)
import jax
import jax.numpy as jnp
from jax import lax
from jax.experimental import pallas as pl
from jax.experimental.pallas import tpu as pltpu
import jax.experimental.pallas.tpu_sc as plsc

_N = 10000
_NP = 10240
_E = 320000
_CB = 128
_NSUB = 16
_NW = 2 * _NSUB
_EPT = _E // _NW
_KC = 80
_GB = 80
_NG = _KC // _GB
_RPT = _NP // _NSUB
_PAD_DST = 10200
_EPS = _E // _NSUB
_KS = 160
_NGS = _KS // _GB


def _make_agg(D, with_deg):
  mesh = plsc.VectorSubcoreMesh(core_axis_name="c", subcore_axis_name="s")
  outs = [jax.ShapeDtypeStruct((2 * _NP, D), jnp.float32)]
  scratch = [
      pltpu.VMEM((_GB, _CB), jnp.int32),
      pltpu.VMEM((_GB, _CB), jnp.int32),
      pltpu.VMEM((_CB, D), jnp.float32),
      pltpu.VMEM((_CB, D), jnp.float32),
      pltpu.VMEM((_CB, D), jnp.float32),
      pltpu.VMEM_SHARED((_NP, D), jnp.float32),
      pltpu.VMEM_SHARED((_NP, D), jnp.float32),
      pltpu.SemaphoreType.DMA,
      pltpu.SemaphoreType.DMA,
      pltpu.SemaphoreType.DMA,
      pltpu.SemaphoreType.DMA,
      pltpu.SemaphoreType.DMA,
      pltpu.SemaphoreType.DMA,
      pltpu.SemaphoreType.DMA,
  ]
  if with_deg:
    outs.append(jax.ShapeDtypeStruct((2 * _NP,), jnp.float32))
    scratch += [
        pltpu.VMEM((_CB,), jnp.float32),
        pltpu.VMEM((_RPT,), jnp.float32),
        pltpu.VMEM_SHARED((_NP,), jnp.float32),
    ]

  def body(*refs):
    if with_deg:
      (y_hbm, src_hbm, dst_hbm, s_hbm, deg_hbm,
       idx_s, idx_d, rows, rows2, rows3, table, acc, sem, sem2, sem3,
       sems_a, sems_b, sems_c, sem_d, ones_v, dstage, dacc) = refs
    else:
      (y_hbm, src_hbm, dst_hbm, s_hbm,
       idx_s, idx_d, rows, rows2, rows3, table, acc, sem, sem2, sem3,
       sems_a, sems_b, sems_c, sem_d) = refs
    c = lax.axis_index("c")
    s = lax.axis_index("s")

    nsub = D // 16
    def _zrow(k, carry):
      rows[k // nsub, pl.ds((k % nsub) * 16, 16)] = jnp.zeros((16,), jnp.float32)
      return carry
    lax.fori_loop(0, _CB * nsub, _zrow, 0)

    base = s * _RPT
    nfull = _RPT // _CB
    for r in range(nfull):
      pltpu.sync_copy(rows, acc.at[pl.ds(base + r * _CB, _CB)])
    pltpu.sync_copy(y_hbm.at[pl.ds(base, _RPT)], table.at[pl.ds(base, _RPT)])

    if with_deg:
      def _fill1(k, carry):
        ones_v[pl.ds(k * 16, 16)] = jnp.ones((16,), jnp.float32)
        return carry
      lax.fori_loop(0, _CB // 16, _fill1, 0)
      def _fillz(k, carry):
        dstage[pl.ds(k * 16, 16)] = jnp.zeros((16,), jnp.float32)
        return carry
      lax.fori_loop(0, _RPT // 16, _fillz, 0)
      pltpu.sync_copy(dstage, dacc.at[pl.ds(base, _RPT)])

    plsc.subcore_barrier()

    def _gather(j, buf, gsem):
      return pltpu.async_copy(table.at[idx_s.at[j]], buf, gsem)

    def _scatter(j, buf, ssem):
      return pltpu.async_copy(buf, acc.at[idx_d.at[j]], ssem, add=True)

    def _deg_scatter(j):
      return pltpu.async_copy(ones_v, dacc.at[idx_d.at[j]], sem_d, add=True)

    def _group(g, carry):
      gbase = c * _KC + g * _GB
      pltpu.sync_copy(src_hbm.at[s, pl.ds(gbase, _GB)], idx_s)
      pltpu.sync_copy(dst_hbm.at[s, pl.ds(gbase, _GB)], idx_d)
      _gather(0, rows, sem)
      _gather(1, rows2, sem2)
      _gather(2, rows3, sem3)
      def _wait_g(j, buf, gsem):
        pltpu.make_async_copy(table.at[idx_s.at[j]], buf, gsem).wait()
      def _triple(p, carry2):
        j0 = 3 * p
        _wait_g(j0, rows, sem)
        sct_a = _scatter(j0, rows, sems_a)
        if with_deg:
          _deg_scatter(j0)
        _wait_g(j0 + 1, rows2, sem2)
        sct_b = _scatter(j0 + 1, rows2, sems_b)
        if with_deg:
          _deg_scatter(j0 + 1)
        _wait_g(j0 + 2, rows3, sem3)
        sct_c = _scatter(j0 + 2, rows3, sems_c)
        if with_deg:
          _deg_scatter(j0 + 2)
        sct_a.wait()
        _gather(j0 + 3, rows, sem)
        sct_b.wait()
        _gather(j0 + 4, rows2, sem2)
        sct_c.wait()
        _gather(j0 + 5, rows3, sem3)
        return carry2
      lax.fori_loop(0, _GB // 3 - 1, _triple, carry)
      j0 = (_GB // 3 - 1) * 3
      _wait_g(j0, rows, sem)
      sct_a = _scatter(j0, rows, sems_a)
      _wait_g(j0 + 1, rows2, sem2)
      sct_b = _scatter(j0 + 1, rows2, sems_b)
      _wait_g(j0 + 2, rows3, sem3)
      sct_c = _scatter(j0 + 2, rows3, sems_c)
      if with_deg:
        _deg_scatter(j0)
        _deg_scatter(j0 + 1)
        _deg_scatter(j0 + 2)
      sct_a.wait()
      _gather(j0 + 3, rows, sem)
      sct_b.wait()
      _gather(j0 + 4, rows2, sem2)
      sct_c.wait()
      _wait_g(j0 + 3, rows, sem)
      sct_a = _scatter(j0 + 3, rows, sems_a)
      _wait_g(j0 + 4, rows2, sem2)
      sct_b = _scatter(j0 + 4, rows2, sems_b)
      if with_deg:
        _deg_scatter(j0 + 3)
        _deg_scatter(j0 + 4)
        for _ in range(_GB):
          pltpu.make_async_copy(ones_v, dacc.at[idx_d.at[0]], sem_d).wait()
      sct_a.wait()
      sct_b.wait()
      return carry
    lax.fori_loop(0, _NG, _group, 0)

    plsc.subcore_barrier()

    obase = c * _NP + s * _RPT
    pltpu.sync_copy(acc.at[pl.ds(base, _RPT)], s_hbm.at[pl.ds(obase, _RPT)])
    if with_deg:
      pltpu.sync_copy(dacc.at[pl.ds(base, _RPT)], dstage)
      pltpu.sync_copy(dstage, deg_hbm.at[pl.ds(obase, _RPT)])

  return pl.kernel(
      body,
      out_type=tuple(outs) if with_deg else outs[0],
      mesh=mesh,
      scratch_types=scratch,
      compiler_params=pltpu.CompilerParams(
          use_tc_tiling_on_sc=False) if D < 128 else None,
  )


def _make_agg_split(with_deg):
  Dh = 64
  mesh = plsc.VectorSubcoreMesh(core_axis_name="c", subcore_axis_name="s")
  outs = [jax.ShapeDtypeStruct((2, _NP, Dh), jnp.float32)]
  scratch = [
      pltpu.VMEM((_GB, _CB), jnp.int32),
      pltpu.VMEM((_GB, _CB), jnp.int32),
      pltpu.VMEM((_CB, Dh), jnp.float32),
      pltpu.VMEM((_CB, Dh), jnp.float32),
      pltpu.VMEM((_CB, Dh), jnp.float32),
      pltpu.VMEM_SHARED((_NP, Dh), jnp.float32),
      pltpu.VMEM_SHARED((_NP, Dh), jnp.float32),
      pltpu.SemaphoreType.DMA,
      pltpu.SemaphoreType.DMA,
      pltpu.SemaphoreType.DMA,
      pltpu.SemaphoreType.DMA,
      pltpu.SemaphoreType.DMA,
      pltpu.SemaphoreType.DMA,
      pltpu.SemaphoreType.DMA,
  ]
  if with_deg:
    outs.append(jax.ShapeDtypeStruct((2 * _NP,), jnp.float32))
    scratch += [
        pltpu.VMEM((_CB,), jnp.float32),
        pltpu.VMEM((_RPT,), jnp.float32),
        pltpu.VMEM_SHARED((_NP,), jnp.float32),
    ]

  def body(*refs):
    if with_deg:
      (y_hbm, src_hbm, dst_hbm, s_hbm, deg_hbm,
       idx_s, idx_d, rows, rows2, rows3, table, acc, sem, sem2, sem3,
       sems_a, sems_b, sems_c, sem_d, ones_v, dstage, dacc) = refs
    else:
      (y_hbm, src_hbm, dst_hbm, s_hbm,
       idx_s, idx_d, rows, rows2, rows3, table, acc, sem, sem2, sem3,
       sems_a, sems_b, sems_c, sem_d) = refs
    c = lax.axis_index("c")
    s = lax.axis_index("s")
    base = s * _RPT
    nfull = _RPT // _CB

    def _zrow(k, carry):
      rows[k // 4, pl.ds((k % 4) * 16, 16)] = jnp.zeros((16,), jnp.float32)
      return carry
    lax.fori_loop(0, _CB * 4, _zrow, 0)
    for r in range(nfull):
      pltpu.sync_copy(rows, acc.at[pl.ds(base + r * _CB, _CB)])
    pltpu.sync_copy(y_hbm.at[c, pl.ds(base, _RPT)], table.at[pl.ds(base, _RPT)])

    if with_deg:
      def _fill1(k, carry):
        ones_v[pl.ds(k * 16, 16)] = jnp.ones((16,), jnp.float32)
        return carry
      lax.fori_loop(0, _CB // 16, _fill1, 0)
      def _fillz(k, carry):
        dstage[pl.ds(k * 16, 16)] = jnp.zeros((16,), jnp.float32)
        return carry
      lax.fori_loop(0, _RPT // 16, _fillz, 0)
      pltpu.sync_copy(dstage, dacc.at[pl.ds(base, _RPT)])

    plsc.subcore_barrier()

    def _gather(j, buf, gsem):
      return pltpu.async_copy(table.at[idx_s.at[j]], buf, gsem)

    def _scatter(j, buf, ssem):
      return pltpu.async_copy(buf, acc.at[idx_d.at[j]], ssem, add=True)

    def _deg_scatter(j):
      return pltpu.async_copy(ones_v, dacc.at[idx_d.at[j]], sem_d, add=True)

    def _group(g, carry):
      pltpu.sync_copy(src_hbm.at[s, pl.ds(g * _GB, _GB)], idx_s)
      pltpu.sync_copy(dst_hbm.at[s, pl.ds(g * _GB, _GB)], idx_d)
      _gather(0, rows, sem)
      _gather(1, rows2, sem2)
      _gather(2, rows3, sem3)
      def _wait_g(j, buf, gsem):
        pltpu.make_async_copy(table.at[idx_s.at[j]], buf, gsem).wait()
      def _triple(p, carry2):
        j0 = 3 * p
        _wait_g(j0, rows, sem)
        sct_a = _scatter(j0, rows, sems_a)
        if with_deg:
          _deg_scatter(j0)
        _wait_g(j0 + 1, rows2, sem2)
        sct_b = _scatter(j0 + 1, rows2, sems_b)
        if with_deg:
          _deg_scatter(j0 + 1)
        _wait_g(j0 + 2, rows3, sem3)
        sct_c = _scatter(j0 + 2, rows3, sems_c)
        if with_deg:
          _deg_scatter(j0 + 2)
        sct_a.wait()
        _gather(j0 + 3, rows, sem)
        sct_b.wait()
        _gather(j0 + 4, rows2, sem2)
        sct_c.wait()
        _gather(j0 + 5, rows3, sem3)
        return carry2
      lax.fori_loop(0, _GB // 3 - 1, _triple, carry)
      j0 = (_GB // 3 - 1) * 3
      _wait_g(j0, rows, sem)
      sct_a = _scatter(j0, rows, sems_a)
      _wait_g(j0 + 1, rows2, sem2)
      sct_b = _scatter(j0 + 1, rows2, sems_b)
      _wait_g(j0 + 2, rows3, sem3)
      sct_c = _scatter(j0 + 2, rows3, sems_c)
      if with_deg:
        _deg_scatter(j0)
        _deg_scatter(j0 + 1)
        _deg_scatter(j0 + 2)
      sct_a.wait()
      _gather(j0 + 3, rows, sem)
      sct_b.wait()
      _gather(j0 + 4, rows2, sem2)
      sct_c.wait()
      _wait_g(j0 + 3, rows, sem)
      sct_a = _scatter(j0 + 3, rows, sems_a)
      _wait_g(j0 + 4, rows2, sem2)
      sct_b = _scatter(j0 + 4, rows2, sems_b)
      if with_deg:
        _deg_scatter(j0 + 3)
        _deg_scatter(j0 + 4)
        for _ in range(_GB):
          pltpu.make_async_copy(ones_v, dacc.at[idx_d.at[0]], sem_d).wait()
      sct_a.wait()
      sct_b.wait()
      return carry
    lax.fori_loop(0, _NGS, _group, 0)

    plsc.subcore_barrier()

    pltpu.sync_copy(acc.at[pl.ds(base, _RPT)], s_hbm.at[c, pl.ds(base, _RPT)])
    if with_deg:
      obase = c * _NP + base
      pltpu.sync_copy(dacc.at[pl.ds(base, _RPT)], dstage)
      pltpu.sync_copy(dstage, deg_hbm.at[pl.ds(obase, _RPT)])

  return pl.kernel(
      body,
      out_type=tuple(outs) if with_deg else outs[0],
      mesh=mesh,
      scratch_types=scratch,
      compiler_params=pltpu.CompilerParams(use_tc_tiling_on_sc=False),
  )


_aggsplit_deg = _make_agg_split(True)
_aggsplit = _make_agg_split(False)
_agg64 = _make_agg(64, False)

_BN = 1024
_GRID = _NP // _BN


def _half_spec(h):
  return pl.BlockSpec((1, _BN, 64), lambda i, h=h: (h, i, 0))


def _row_spec(d):
  return pl.BlockSpec((_BN, d), lambda i: (i, 0))


def _row_spec_hi(d):
  return pl.BlockSpec((_BN, d), lambda i: (i + _GRID, 0))


def _full_spec(r, c):
  return pl.BlockSpec((r, c), lambda i: (0, 0))


def _invd1(dg_ref):
  return 1.0 / jnp.maximum(dg_ref[...], 1.0)


def _tc_b_body(s0a, s0b, dg, w0t, b0, w1t, out):
  s0 = jnp.concatenate([s0a[0], s0b[0]], axis=-1)
  agg = s0 * _invd1(dg)
  h0 = jnp.dot(agg, w0t[...], preferred_element_type=jnp.float32) + b0[...]
  h0 = jnp.maximum(h0, 0.0)
  y1 = jnp.dot(h0, w1t[...], preferred_element_type=jnp.float32)
  out[0] = y1[:, :64]
  out[1] = y1[:, 64:]


def _tc_c_body(s1a, s1b, dg, b1, w2at, w2bt, out):
  s1 = jnp.concatenate([s1a[0], s1b[0]], axis=-1)
  t = s1 * _invd1(dg) + b1[...]
  z = jnp.dot(t, w2at[...], preferred_element_type=jnp.float32)
  z = z + jnp.dot(jnp.maximum(t, 0.0), w2bt[...],
                  preferred_element_type=jnp.float32)
  out[...] = z


def _tc_d_body(s2a, s2b, dg, b2p, out):
  out[...] = (s2a[...] + s2b[...]) * _invd1(dg) + b2p[...]


def kernel(x, edge_index, W0, b0, W1, b1, W2, b2):
  pad_s = _KS * _CB - _EPS
  src_s = jnp.pad(edge_index[0].reshape(_NSUB, _EPS), ((0, 0), (0, pad_s)),
                  constant_values=0).reshape(_NSUB, _KS, _CB)
  dst_s = jnp.pad(edge_index[1].reshape(_NSUB, _EPS), ((0, 0), (0, pad_s)),
                  constant_values=_PAD_DST).reshape(_NSUB, _KS, _CB)
  x3 = jnp.pad(jnp.stack([x[:, :64], x[:, 64:]], axis=0),
               ((0, 0), (0, _NP - _N), (0, 0)))

  S0, degp = _aggsplit_deg(x3, src_s, dst_s)
  degc = degp[:_NP].reshape(_NP, 1)

  y3 = pl.pallas_call(
      _tc_b_body,
      grid=(_GRID,),
      in_specs=[_half_spec(0), _half_spec(1), _row_spec(1),
                _full_spec(128, 128), _full_spec(1, 128),
                _full_spec(128, 128)],
      out_specs=pl.BlockSpec((2, _BN, 64), lambda i: (0, i, 0)),
      out_shape=jax.ShapeDtypeStruct((2, _NP, 64), jnp.float32),
  )(S0, S0, degc, W0.T, b0.reshape(1, -1), W1.T)

  S1 = _aggsplit(y3, src_s, dst_s)

  W2p = jnp.pad(W2, ((0, 64 - W2.shape[0]), (0, 0)))
  z = pl.pallas_call(
      _tc_c_body,
      grid=(_GRID,),
      in_specs=[_half_spec(0), _half_spec(1), _row_spec(1),
                _full_spec(1, 128), _full_spec(128, 64),
                _full_spec(128, 64)],
      out_specs=_row_spec(64),
      out_shape=jax.ShapeDtypeStruct((_NP, 64), jnp.float32),
  )(S1, S1, degc, b1.reshape(1, -1), W2p[:, :128].T, W2p[:, 128:].T)

  S2 = _agg64(z, src_s, dst_s)

  b2p = jnp.pad(b2, (0, 64 - b2.shape[0]))
  out = pl.pallas_call(
      _tc_d_body,
      grid=(_GRID,),
      in_specs=[_row_spec(64), _row_spec_hi(64), _row_spec(1),
                _full_spec(1, 64)],
      out_specs=_row_spec(64),
      out_shape=jax.ShapeDtypeStruct((_NP, 64), jnp.float32),
  )(S2, S2, degc, b2p.reshape(1, -1))

  return out[:_N, :41]


# speedup vs baseline: 1.1228x; 1.1228x over previous
import jax
import jax.numpy as jnp
from jax import lax
from jax.experimental import pallas as pl
from jax.experimental.pallas import tpu as pltpu
import jax.experimental.pallas.tpu_sc as plsc

_N = 10000
_NP = 10240
_E = 320000
_CB = 128
_NSUB = 16
_NW = 2 * _NSUB
_EPT = _E // _NW
_KC = 80
_GB = 80
_NG = _KC // _GB
_RPT = _NP // _NSUB
_PAD_DST = 10200
_EPS = _E // _NSUB
_KS = 160
_NGS = _KS // _GB


def _make_agg(D, with_deg):
  mesh = plsc.VectorSubcoreMesh(core_axis_name="c", subcore_axis_name="s")
  outs = [jax.ShapeDtypeStruct((2 * _NP, D), jnp.float32)]
  scratch = [
      pltpu.VMEM((_GB, _CB), jnp.int32),
      pltpu.VMEM((_GB, _CB), jnp.int32),
      pltpu.VMEM((_CB, D), jnp.float32),
      pltpu.VMEM((_CB, D), jnp.float32),
      pltpu.VMEM_SHARED((_NP, D), jnp.float32),
      pltpu.VMEM_SHARED((_NP, D), jnp.float32),
      pltpu.SemaphoreType.DMA,
      pltpu.SemaphoreType.DMA,
      pltpu.SemaphoreType.DMA,
      pltpu.SemaphoreType.DMA,
      pltpu.SemaphoreType.DMA,
  ]
  if with_deg:
    outs.append(jax.ShapeDtypeStruct((2 * _NP,), jnp.float32))
    scratch += [
        pltpu.VMEM((_CB,), jnp.float32),
        pltpu.VMEM((_RPT,), jnp.float32),
        pltpu.VMEM_SHARED((_NP,), jnp.float32),
    ]

  def body(*refs):
    if with_deg:
      (y_hbm, src_hbm, dst_hbm, s_hbm, deg_hbm,
       idx_s, idx_d, rows, rows2, table, acc, sem, sem2, sems_a, sems_b,
       sem_d, ones_v, dstage, dacc) = refs
    else:
      (y_hbm, src_hbm, dst_hbm, s_hbm,
       idx_s, idx_d, rows, rows2, table, acc, sem, sem2, sems_a, sems_b,
       sem_d) = refs
    c = lax.axis_index("c")
    s = lax.axis_index("s")

    nsub = D // 16
    def _zrow(k, carry):
      rows[k // nsub, pl.ds((k % nsub) * 16, 16)] = jnp.zeros((16,), jnp.float32)
      return carry
    lax.fori_loop(0, _CB * nsub, _zrow, 0)

    base = s * _RPT
    nfull = _RPT // _CB
    for r in range(nfull):
      pltpu.sync_copy(rows, acc.at[pl.ds(base + r * _CB, _CB)])
    pltpu.sync_copy(y_hbm.at[pl.ds(base, _RPT)], table.at[pl.ds(base, _RPT)])

    if with_deg:
      def _fill1(k, carry):
        ones_v[pl.ds(k * 16, 16)] = jnp.ones((16,), jnp.float32)
        return carry
      lax.fori_loop(0, _CB // 16, _fill1, 0)
      def _fillz(k, carry):
        dstage[pl.ds(k * 16, 16)] = jnp.zeros((16,), jnp.float32)
        return carry
      lax.fori_loop(0, _RPT // 16, _fillz, 0)
      pltpu.sync_copy(dstage, dacc.at[pl.ds(base, _RPT)])

    plsc.subcore_barrier()

    def _gather(j, buf, gsem):
      return pltpu.async_copy(table.at[idx_s.at[j]], buf, gsem)

    def _scatter(j, buf, ssem):
      return pltpu.async_copy(buf, acc.at[idx_d.at[j]], ssem, add=True)

    def _deg_scatter(j):
      return pltpu.async_copy(ones_v, dacc.at[idx_d.at[j]], sem_d, add=True)

    def _group(g, carry):
      gbase = c * _KC + g * _GB
      pltpu.sync_copy(src_hbm.at[s, pl.ds(gbase, _GB)], idx_s)
      pltpu.sync_copy(dst_hbm.at[s, pl.ds(gbase, _GB)], idx_d)
      _gather(0, rows, sem)
      _gather(1, rows2, sem2)
      def _pair(p, carry2):
        j0 = 2 * p
        pltpu.make_async_copy(table.at[idx_s.at[j0]], rows, sem).wait()
        sct_a = _scatter(j0, rows, sems_a)
        if with_deg:
          _deg_scatter(j0)
        pltpu.make_async_copy(table.at[idx_s.at[j0 + 1]], rows2, sem2).wait()
        sct_b = _scatter(j0 + 1, rows2, sems_b)
        if with_deg:
          _deg_scatter(j0 + 1)
        sct_a.wait()
        _gather(j0 + 2, rows, sem)
        sct_b.wait()
        _gather(j0 + 3, rows2, sem2)
        return carry2
      lax.fori_loop(0, _GB // 2 - 1, _pair, carry)
      j0 = _GB - 2
      pltpu.make_async_copy(table.at[idx_s.at[j0]], rows, sem).wait()
      sct_a = _scatter(j0, rows, sems_a)
      pltpu.make_async_copy(table.at[idx_s.at[j0 + 1]], rows2, sem2).wait()
      sct_b = _scatter(j0 + 1, rows2, sems_b)
      if with_deg:
        _deg_scatter(j0)
        _deg_scatter(j0 + 1)
        for _ in range(_GB):
          pltpu.make_async_copy(ones_v, dacc.at[idx_d.at[0]], sem_d).wait()
      sct_a.wait()
      sct_b.wait()
      return carry
    lax.fori_loop(0, _NG, _group, 0)

    plsc.subcore_barrier()

    obase = c * _NP + s * _RPT
    pltpu.sync_copy(acc.at[pl.ds(base, _RPT)], s_hbm.at[pl.ds(obase, _RPT)])
    if with_deg:
      pltpu.sync_copy(dacc.at[pl.ds(base, _RPT)], dstage)
      pltpu.sync_copy(dstage, deg_hbm.at[pl.ds(obase, _RPT)])

  return pl.kernel(
      body,
      out_type=tuple(outs) if with_deg else outs[0],
      mesh=mesh,
      scratch_types=scratch,
      compiler_params=pltpu.CompilerParams(
          use_tc_tiling_on_sc=False) if D < 128 else None,
  )


def _make_agg_split(with_deg):
  Dh = 64
  mesh = plsc.VectorSubcoreMesh(core_axis_name="c", subcore_axis_name="s")
  outs = [jax.ShapeDtypeStruct((2, _NP, Dh), jnp.float32)]
  scratch = [
      pltpu.VMEM((_GB, _CB), jnp.int32),
      pltpu.VMEM((_GB, _CB), jnp.int32),
      pltpu.VMEM((_CB, Dh), jnp.float32),
      pltpu.VMEM((_CB, Dh), jnp.float32),
      pltpu.VMEM_SHARED((_NP, Dh), jnp.float32),
      pltpu.VMEM_SHARED((_NP, Dh), jnp.float32),
      pltpu.SemaphoreType.DMA,
      pltpu.SemaphoreType.DMA,
      pltpu.SemaphoreType.DMA,
      pltpu.SemaphoreType.DMA,
      pltpu.SemaphoreType.DMA,
  ]
  if with_deg:
    outs.append(jax.ShapeDtypeStruct((2 * _NP,), jnp.float32))
    scratch += [
        pltpu.VMEM((_CB,), jnp.float32),
        pltpu.VMEM((_RPT,), jnp.float32),
        pltpu.VMEM_SHARED((_NP,), jnp.float32),
    ]

  def body(*refs):
    if with_deg:
      (y_hbm, src_hbm, dst_hbm, s_hbm, deg_hbm,
       idx_s, idx_d, rows, rows2, table, acc, sem, sem2, sems_a, sems_b,
       sem_d, ones_v, dstage, dacc) = refs
    else:
      (y_hbm, src_hbm, dst_hbm, s_hbm,
       idx_s, idx_d, rows, rows2, table, acc, sem, sem2, sems_a, sems_b,
       sem_d) = refs
    c = lax.axis_index("c")
    s = lax.axis_index("s")
    base = s * _RPT
    nfull = _RPT // _CB

    def _zrow(k, carry):
      rows[k // 4, pl.ds((k % 4) * 16, 16)] = jnp.zeros((16,), jnp.float32)
      return carry
    lax.fori_loop(0, _CB * 4, _zrow, 0)
    for r in range(nfull):
      pltpu.sync_copy(rows, acc.at[pl.ds(base + r * _CB, _CB)])
    pltpu.sync_copy(y_hbm.at[c, pl.ds(base, _RPT)], table.at[pl.ds(base, _RPT)])

    if with_deg:
      def _fill1(k, carry):
        ones_v[pl.ds(k * 16, 16)] = jnp.ones((16,), jnp.float32)
        return carry
      lax.fori_loop(0, _CB // 16, _fill1, 0)
      def _fillz(k, carry):
        dstage[pl.ds(k * 16, 16)] = jnp.zeros((16,), jnp.float32)
        return carry
      lax.fori_loop(0, _RPT // 16, _fillz, 0)
      pltpu.sync_copy(dstage, dacc.at[pl.ds(base, _RPT)])

    plsc.subcore_barrier()

    def _gather(j, buf, gsem):
      return pltpu.async_copy(table.at[idx_s.at[j]], buf, gsem)

    def _scatter(j, buf, ssem):
      return pltpu.async_copy(buf, acc.at[idx_d.at[j]], ssem, add=True)

    def _deg_scatter(j):
      return pltpu.async_copy(ones_v, dacc.at[idx_d.at[j]], sem_d, add=True)

    def _group(g, carry):
      pltpu.sync_copy(src_hbm.at[s, pl.ds(g * _GB, _GB)], idx_s)
      pltpu.sync_copy(dst_hbm.at[s, pl.ds(g * _GB, _GB)], idx_d)
      _gather(0, rows, sem)
      _gather(1, rows2, sem2)
      def _pair(p, carry2):
        j0 = 2 * p
        pltpu.make_async_copy(table.at[idx_s.at[j0]], rows, sem).wait()
        sct_a = _scatter(j0, rows, sems_a)
        if with_deg:
          _deg_scatter(j0)
        pltpu.make_async_copy(table.at[idx_s.at[j0 + 1]], rows2, sem2).wait()
        sct_b = _scatter(j0 + 1, rows2, sems_b)
        if with_deg:
          _deg_scatter(j0 + 1)
        sct_a.wait()
        _gather(j0 + 2, rows, sem)
        sct_b.wait()
        _gather(j0 + 3, rows2, sem2)
        return carry2
      lax.fori_loop(0, _GB // 2 - 1, _pair, carry)
      j0 = _GB - 2
      pltpu.make_async_copy(table.at[idx_s.at[j0]], rows, sem).wait()
      sct_a = _scatter(j0, rows, sems_a)
      pltpu.make_async_copy(table.at[idx_s.at[j0 + 1]], rows2, sem2).wait()
      sct_b = _scatter(j0 + 1, rows2, sems_b)
      if with_deg:
        _deg_scatter(j0)
        _deg_scatter(j0 + 1)
        for _ in range(_GB):
          pltpu.make_async_copy(ones_v, dacc.at[idx_d.at[0]], sem_d).wait()
      sct_a.wait()
      sct_b.wait()
      return carry
    lax.fori_loop(0, _NGS, _group, 0)

    plsc.subcore_barrier()

    pltpu.sync_copy(acc.at[pl.ds(base, _RPT)], s_hbm.at[c, pl.ds(base, _RPT)])
    if with_deg:
      obase = c * _NP + base
      pltpu.sync_copy(dacc.at[pl.ds(base, _RPT)], dstage)
      pltpu.sync_copy(dstage, deg_hbm.at[pl.ds(obase, _RPT)])

  return pl.kernel(
      body,
      out_type=tuple(outs) if with_deg else outs[0],
      mesh=mesh,
      scratch_types=scratch,
      compiler_params=pltpu.CompilerParams(use_tc_tiling_on_sc=False),
  )


_aggsplit_deg = _make_agg_split(True)
_aggsplit = _make_agg_split(False)
_agg64 = _make_agg(64, False)

_BN = 1024
_GRID = _NP // _BN


def _half_spec(h):
  return pl.BlockSpec((1, _BN, 64), lambda i, h=h: (h, i, 0))


def _row_spec(d):
  return pl.BlockSpec((_BN, d), lambda i: (i, 0))


def _row_spec_hi(d):
  return pl.BlockSpec((_BN, d), lambda i: (i + _GRID, 0))


def _full_spec(r, c):
  return pl.BlockSpec((r, c), lambda i: (0, 0))


def _invd1(dg_ref):
  return 1.0 / jnp.maximum(dg_ref[...], 1.0)


def _tc_b_body(s0a, s0b, dg, w0t, b0, w1t, out):
  s0 = jnp.concatenate([s0a[0], s0b[0]], axis=-1)
  agg = s0 * _invd1(dg)
  h0 = jnp.dot(agg, w0t[...], preferred_element_type=jnp.float32) + b0[...]
  h0 = jnp.maximum(h0, 0.0)
  y1 = jnp.dot(h0, w1t[...], preferred_element_type=jnp.float32)
  out[0] = y1[:, :64]
  out[1] = y1[:, 64:]


def _tc_c_body(s1a, s1b, dg, b1, w2at, w2bt, out):
  s1 = jnp.concatenate([s1a[0], s1b[0]], axis=-1)
  t = s1 * _invd1(dg) + b1[...]
  z = jnp.dot(t, w2at[...], preferred_element_type=jnp.float32)
  z = z + jnp.dot(jnp.maximum(t, 0.0), w2bt[...],
                  preferred_element_type=jnp.float32)
  out[...] = z


def _tc_d_body(s2a, s2b, dg, b2p, out):
  out[...] = (s2a[...] + s2b[...]) * _invd1(dg) + b2p[...]


def kernel(x, edge_index, W0, b0, W1, b1, W2, b2):
  pad_s = _KS * _CB - _EPS
  src_s = jnp.pad(edge_index[0].reshape(_NSUB, _EPS), ((0, 0), (0, pad_s)),
                  constant_values=0).reshape(_NSUB, _KS, _CB)
  dst_s = jnp.pad(edge_index[1].reshape(_NSUB, _EPS), ((0, 0), (0, pad_s)),
                  constant_values=_PAD_DST).reshape(_NSUB, _KS, _CB)
  x3 = jnp.pad(jnp.stack([x[:, :64], x[:, 64:]], axis=0),
               ((0, 0), (0, _NP - _N), (0, 0)))

  S0, degp = _aggsplit_deg(x3, src_s, dst_s)
  degc = degp[:_NP].reshape(_NP, 1)

  y3 = pl.pallas_call(
      _tc_b_body,
      grid=(_GRID,),
      in_specs=[_half_spec(0), _half_spec(1), _row_spec(1),
                _full_spec(128, 128), _full_spec(1, 128),
                _full_spec(128, 128)],
      out_specs=pl.BlockSpec((2, _BN, 64), lambda i: (0, i, 0)),
      out_shape=jax.ShapeDtypeStruct((2, _NP, 64), jnp.float32),
  )(S0, S0, degc, W0.T, b0.reshape(1, -1), W1.T)

  S1 = _aggsplit(y3, src_s, dst_s)

  W2p = jnp.pad(W2, ((0, 64 - W2.shape[0]), (0, 0)))
  z = pl.pallas_call(
      _tc_c_body,
      grid=(_GRID,),
      in_specs=[_half_spec(0), _half_spec(1), _row_spec(1),
                _full_spec(1, 128), _full_spec(128, 64),
                _full_spec(128, 64)],
      out_specs=_row_spec(64),
      out_shape=jax.ShapeDtypeStruct((_NP, 64), jnp.float32),
  )(S1, S1, degc, b1.reshape(1, -1), W2p[:, :128].T, W2p[:, 128:].T)

  S2 = _agg64(z, src_s, dst_s)

  b2p = jnp.pad(b2, (0, 64 - b2.shape[0]))
  out = pl.pallas_call(
      _tc_d_body,
      grid=(_GRID,),
      in_specs=[_row_spec(64), _row_spec_hi(64), _row_spec(1),
                _full_spec(1, 64)],
      out_specs=_row_spec(64),
      out_shape=jax.ShapeDtypeStruct((_NP, 64), jnp.float32),
  )(S2, S2, degc, b2p.reshape(1, -1))

  return out[:_N, :41]
